# baseline (speedup 1.0000x reference)
import jax
import jax.numpy as jnp
from jax.experimental import pallas as pl
from jax.experimental.pallas import tpu as pltpu

N = 16384
F = 128
K = 1000
KP = 1024
BN = 1024


def _kmeans_block(x_ref, ctm2_ref, csq_ref, dist_ref, asn_ref, inr_ref):
    x = x_ref[...]
    x_sq = jnp.sum(x * x, axis=1, keepdims=True)
    cross = jnp.dot(x, ctm2_ref[...], preferred_element_type=jnp.float32)
    d2 = jnp.maximum(x_sq + (csq_ref[...] + cross), 0.0)
    m2 = jnp.min(d2, axis=1, keepdims=True)
    dist_ref[...] = jnp.sqrt(d2[:, :K])
    idx = jax.lax.broadcasted_iota(jnp.int32, d2.shape, 1)
    asn_ref[...] = jnp.min(jnp.where(d2 == m2, idx, KP), axis=1)
    inr_ref[...] = m2[:, 0]


@jax.jit
def kernel(data, centroids):
    ctm2 = jnp.concatenate(
        [-2.0 * centroids.T, jnp.zeros((F, KP - K), jnp.float32)], axis=1)
    csq = jnp.concatenate(
        [jnp.sum(centroids * centroids, axis=1),
         jnp.full((KP - K,), jnp.inf, jnp.float32)])[None, :]
    grid = (N // BN,)
    distances, assignments, inertias = pl.pallas_call(
        _kmeans_block,
        grid=grid,
        in_specs=[
            pl.BlockSpec((BN, F), lambda i: (i, 0)),
            pl.BlockSpec((F, KP), lambda i: (0, 0)),
            pl.BlockSpec((1, KP), lambda i: (0, 0)),
        ],
        out_specs=[
            pl.BlockSpec((BN, K), lambda i: (i, 0)),
            pl.BlockSpec((BN,), lambda i: (i,)),
            pl.BlockSpec((BN,), lambda i: (i,)),
        ],
        out_shape=[
            jax.ShapeDtypeStruct((N, K), jnp.float32),
            jax.ShapeDtypeStruct((N,), jnp.int32),
            jax.ShapeDtypeStruct((N,), jnp.float32),
        ],
        compiler_params=pltpu.CompilerParams(
            dimension_semantics=("arbitrary",),
        ),
    )(data, ctm2, csq)
    return (distances, assignments, inertias)


# speedup vs baseline: 1.2663x; 1.2663x over previous
import jax
import jax.numpy as jnp
from jax.experimental import pallas as pl
from jax.experimental.pallas import tpu as pltpu

N = 16384
F = 128
K = 1000
KP = 1024
BN = 1024


def _kmeans_block(x_ref, ctm2_ref, csq_ref, dist_ref, asn_ref, inr_ref):
    x = x_ref[...]
    x_sq = jnp.sum(x * x, axis=1, keepdims=True)
    cross = jnp.dot(x, ctm2_ref[...], preferred_element_type=jnp.float32)
    d2 = jnp.maximum(x_sq + (csq_ref[...] + cross), 1e-12)
    dist_ref[...] = (d2 * jax.lax.rsqrt(d2))[:, :K]
    m2 = jnp.min(d2, axis=1, keepdims=True)
    idx = jax.lax.broadcasted_iota(jnp.int32, d2.shape, 1)
    asn_ref[...] = jnp.min(jnp.where(d2 == m2, idx, KP), axis=1)
    inr_ref[...] = m2[:, 0]


@jax.jit
def kernel(data, centroids):
    ctm2 = jnp.concatenate(
        [-2.0 * centroids.T, jnp.zeros((F, KP - K), jnp.float32)], axis=1)
    csq = jnp.concatenate(
        [jnp.sum(centroids * centroids, axis=1),
         jnp.full((KP - K,), 1e30, jnp.float32)])[None, :]
    grid = (N // BN,)
    distances, assignments, inertias = pl.pallas_call(
        _kmeans_block,
        grid=grid,
        in_specs=[
            pl.BlockSpec((BN, F), lambda i: (i, 0)),
            pl.BlockSpec((F, KP), lambda i: (0, 0)),
            pl.BlockSpec((1, KP), lambda i: (0, 0)),
        ],
        out_specs=[
            pl.BlockSpec((BN, K), lambda i: (i, 0)),
            pl.BlockSpec((BN,), lambda i: (i,)),
            pl.BlockSpec((BN,), lambda i: (i,)),
        ],
        out_shape=[
            jax.ShapeDtypeStruct((N, K), jnp.float32),
            jax.ShapeDtypeStruct((N,), jnp.int32),
            jax.ShapeDtypeStruct((N,), jnp.float32),
        ],
        compiler_params=pltpu.CompilerParams(
            dimension_semantics=("arbitrary",),
        ),
    )(data, ctm2, csq)
    return (distances, assignments, inertias)


# speedup vs baseline: 1.2927x; 1.0208x over previous
import jax
import jax.numpy as jnp
from jax.experimental import pallas as pl
from jax.experimental.pallas import tpu as pltpu

N = 16384
F = 128
K = 1000
KP = 1024
BN = 2048


def _kmeans_block(x_ref, ctm2_ref, csq_ref, dist_ref, asn_ref, inr_ref):
    x = x_ref[...]
    x_sq = jnp.sum(x * x, axis=1, keepdims=True)
    cross = jnp.dot(x, ctm2_ref[...], preferred_element_type=jnp.float32)
    d2 = jnp.maximum(x_sq + (csq_ref[...] + cross), 1e-12)
    dist_ref[...] = (d2 * jax.lax.rsqrt(d2))[:, :K]
    m2 = jnp.min(d2, axis=1, keepdims=True)
    idx = jax.lax.broadcasted_iota(jnp.int32, d2.shape, 1)
    asn_ref[...] = jnp.min(jnp.where(d2 == m2, idx, KP), axis=1,
                           keepdims=True)
    inr_ref[...] = m2


@jax.jit
def kernel(data, centroids):
    ctm2 = jnp.concatenate(
        [-2.0 * centroids.T, jnp.zeros((F, KP - K), jnp.float32)], axis=1)
    csq = jnp.concatenate(
        [jnp.sum(centroids * centroids, axis=1),
         jnp.full((KP - K,), 1e30, jnp.float32)])[None, :]
    grid = (N // BN,)
    distances, assignments, inertias = pl.pallas_call(
        _kmeans_block,
        grid=grid,
        in_specs=[
            pl.BlockSpec((BN, F), lambda i: (i, 0)),
            pl.BlockSpec((F, KP), lambda i: (0, 0)),
            pl.BlockSpec((1, KP), lambda i: (0, 0)),
        ],
        out_specs=[
            pl.BlockSpec((BN, K), lambda i: (i, 0)),
            pl.BlockSpec((BN, 1), lambda i: (i, 0)),
            pl.BlockSpec((BN, 1), lambda i: (i, 0)),
        ],
        out_shape=[
            jax.ShapeDtypeStruct((N, K), jnp.float32),
            jax.ShapeDtypeStruct((N, 1), jnp.int32),
            jax.ShapeDtypeStruct((N, 1), jnp.float32),
        ],
        compiler_params=pltpu.CompilerParams(
            dimension_semantics=("arbitrary",),
        ),
    )(data, ctm2, csq)
    return (distances, assignments[:, 0], inertias[:, 0])
